# baseline (speedup 1.0000x reference)
import functools

import jax
import jax.numpy as jnp
from jax import lax
from jax.experimental import pallas as pl
from jax.experimental.pallas import tpu as pltpu
from jax.experimental.pallas import tpu_sc as plsc

NC = 2
NS = 16
NW = NC * NS
CH = 128
SEG = 20


def _proj1_body(x_ref, w_ref, b_ref, p_ref, r_ref):
    res = jnp.dot(x_ref[...], w_ref[...], preferred_element_type=jnp.float32)
    p_ref[...] = res[:, :16]
    r_ref[...] = res[:, 16:] + b_ref[...]


def _recip_cols(degp_ref, n, width):
    deg2 = degp_ref[0:1, :n] + degp_ref[1:2, :n]
    rec = 1.0 / jnp.maximum(deg2, 1.0)
    return jax.lax.dot_general(
        rec, jnp.ones((1, width), jnp.float32),
        (((0,), (0,)), ((), ())), preferred_element_type=jnp.float32)


def _layer1_epilogue_body(aggp_ref, degp_ref, r_ref, h_ref):
    n = r_ref.shape[0]
    agg = aggp_ref[0, :n, :] + aggp_ref[1, :n, :]
    mean = agg * _recip_cols(degp_ref, n, agg.shape[1])
    h_ref[...] = jnp.maximum(mean + r_ref[...], 0.0)


def _layer2_body(aggp_ref, degp_ref, h_ref, wl_ref, wr_ref, b_ref,
                 logp_ref, out_ref):
    n = h_ref.shape[0]
    agg = aggp_ref[0, :n, :] + aggp_ref[1, :n, :]
    mean = agg * _recip_cols(degp_ref, n, agg.shape[1])
    out = (jnp.dot(mean, wl_ref[...], preferred_element_type=jnp.float32)
           + jnp.dot(h_ref[...], wr_ref[...],
                     preferred_element_type=jnp.float32)
           + b_ref[...])
    out_ref[...] = out
    shifted = out - jnp.max(out, axis=1, keepdims=True)
    logp_ref[...] = shifted - jnp.log(
        jnp.sum(jnp.exp(shifted), axis=1, keepdims=True))


def _make_segsum(n3_rows, n_chunks, n_real, n_pad_edges, with_deg,
                 stage_table):
    rows_pt = n3_rows // NS

    mesh = plsc.VectorSubcoreMesh(core_axis_name="c", subcore_axis_name="s")
    out_type = [jax.ShapeDtypeStruct((NC, n3_rows, 16), jnp.float32)]
    if with_deg:
        out_type.append(jax.ShapeDtypeStruct((NC, n3_rows), jnp.float32))
    n_segs = n_chunks // SEG
    scratch = [
        pltpu.VMEM((n_chunks, CH), jnp.int32),
        pltpu.VMEM((n_chunks, CH), jnp.int32),
        [pltpu.VMEM((SEG * CH, 16), jnp.float32) for _ in range(2)],
        pltpu.VMEM((rows_pt, 16), jnp.float32),
        pltpu.VMEM_SHARED((n3_rows, 16), jnp.float32),
        [pltpu.SemaphoreType.DMA for _ in range(2)],
        [pltpu.SemaphoreType.DMA for _ in range(2)],
    ]
    if stage_table:
        scratch.append(pltpu.VMEM_SHARED((n3_rows, 16), jnp.float32))
    if with_deg:
        scratch += [
            pltpu.VMEM((CH,), jnp.float32),
            pltpu.VMEM((-(-rows_pt // 16) * 16,), jnp.float32),
            pltpu.VMEM_SHARED((n3_rows,), jnp.float32),
            [pltpu.SemaphoreType.DMA for _ in range(2)],
        ]

    def body(table_h, ei_h, ones1_h, *rest):
        it = iter(rest)
        agg_out = next(it)
        deg_out = next(it) if with_deg else None
        srcv, dstv, bufs, bounce, agg_s, gsem, ssem = (
            next(it), next(it), next(it), next(it), next(it), next(it),
            next(it))
        table_sc = next(it) if stage_table else None
        if with_deg:
            ones1v, degv, deg_s, dsem = next(it), next(it), next(it), next(it)
        cid = lax.axis_index("c")
        sid = lax.axis_index("s")
        wid = sid * NC + cid
        t0 = sid * rows_pt
        if stage_table:
            pltpu.sync_copy(table_h.at[pl.ds(t0, rows_pt)], bounce)
            pltpu.sync_copy(bounce, table_sc.at[pl.ds(t0, rows_pt)])
            table_s = table_sc
        else:
            table_s = table_h
        zv = jnp.zeros((16,), jnp.float32)

        def _zrow(r, c2):
            bounce[r, :] = zv
            return c2

        lax.fori_loop(0, rows_pt, _zrow, 0)
        pltpu.sync_copy(bounce, agg_s.at[pl.ds(t0, rows_pt)])
        if with_deg:
            def _zdeg(i, c2):
                degv[pl.ds(i * 16, 16)] = zv
                return c2

            lax.fori_loop(0, degv.shape[0] // 16, _zdeg, 0)
            pltpu.sync_copy(degv.at[pl.ds(0, rows_pt)],
                            deg_s.at[pl.ds(t0, rows_pt)])
            pltpu.sync_copy(ones1_h, ones1v)
        pltpu.sync_copy(ei_h.at[0, wid], srcv)
        pltpu.sync_copy(ei_h.at[1, wid], dstv)
        pad_rows = n_pad_edges // CH
        if pad_rows:
            iot = lax.iota(jnp.int32, 16)
            dpatt = n_real + iot

            @pl.when(wid == NW - 1)
            def _():
                for r in range(n_chunks - pad_rows, n_chunks):
                    for j in range(CH // 16):
                        dstv[r, pl.ds(j * 16, 16)] = dpatt
                        srcv[r, pl.ds(j * 16, 16)] = (
                            ((r * CH + j * 16) * 131 + iot * 7) % n_real)
        plsc.subcore_barrier()

        def fire_gathers(s, b):
            for k in range(SEG):
                pltpu.async_copy(table_s.at[srcv.at[s * SEG + k]],
                                 bufs[b].at[pl.ds(k * CH, CH)], gsem[b])

        def drain(sem, src, dst):
            for _ in range(SEG):
                pltpu.make_async_copy(src, dst, sem).wait()

        def fire_scatters(s, b):
            for k in range(SEG):
                c = s * SEG + k
                pltpu.async_copy(bufs[b].at[pl.ds(k * CH, CH)],
                                 agg_s.at[dstv.at[c]], ssem[b], add=True)
                if with_deg:
                    pltpu.async_copy(ones1v, deg_s.at[dstv.at[c]],
                                     dsem[b], add=True)

        def drain_scatters(b):
            drain(ssem[b], bufs[b].at[pl.ds(0, CH)], agg_s.at[dstv.at[0]])
            if with_deg:
                drain(dsem[b], ones1v, deg_s.at[dstv.at[0]])

        fire_gathers(0, 0)
        for s in range(n_segs):
            b = s % 2
            if s > 0:
                drain_scatters(1 - b)
            if s + 1 < n_segs:
                fire_gathers(s + 1, 1 - b)
            drain(gsem[b], table_s.at[srcv.at[0]], bufs[b].at[pl.ds(0, CH)])
            fire_scatters(s, b)
        drain_scatters((n_segs - 1) % 2)
        plsc.subcore_barrier()
        pltpu.sync_copy(agg_s.at[pl.ds(t0, rows_pt)], bounce)
        pltpu.sync_copy(bounce, agg_out.at[cid, pl.ds(t0, rows_pt)])
        if with_deg:
            pltpu.sync_copy(deg_s.at[pl.ds(t0, rows_pt)],
                            degv.at[pl.ds(0, rows_pt)])
            pltpu.sync_copy(degv.at[pl.ds(0, rows_pt)],
                            deg_out.at[cid, pl.ds(t0, rows_pt)])

    return pl.kernel(body, out_type=out_type, mesh=mesh,
                     scratch_types=scratch,
                     compiler_params=pltpu.CompilerParams(
                         use_tc_tiling_on_sc=False))


def kernel(x, edge_index, W1l, b1l, W1r, W2l, b2l, W2r):
    n, f_in = x.shape
    dim = W1l.shape[0]
    c_out = W2l.shape[0]
    e = edge_index.shape[1]
    assert dim == 16

    n3 = -(-(n + 16) // 128) * 128
    n_chunks = -(-e // (NW * CH))
    n_chunks = -(-n_chunks // SEG) * SEG
    npad = n_chunks * NW * CH - e

    ei_p = jnp.pad(edge_index, ((0, 0), (0, npad)),
                   constant_values=n).reshape(2, NW, n_chunks, CH)
    w1 = jnp.concatenate([W1l, W1r], axis=0).T
    b1 = b1l.reshape(1, dim)
    w2l_t = W2l.T
    w2r_t = W2r.T
    b2 = b2l.reshape(1, c_out)
    ones1 = jnp.ones((CH,), jnp.float32)

    p1, r1 = pl.pallas_call(
        _proj1_body,
        out_shape=(jax.ShapeDtypeStruct((n, dim), jnp.float32),
                   jax.ShapeDtypeStruct((n, dim), jnp.float32)),
    )(x, w1, b1)

    p1_pad = jnp.pad(p1, ((0, n3 - n), (0, 0)))
    segsum_deg = _make_segsum(n3, n_chunks, n, npad, with_deg=True,
                              stage_table=False)
    aggp1, degp = segsum_deg(p1_pad, ei_p, ones1)

    h = pl.pallas_call(
        _layer1_epilogue_body,
        out_shape=jax.ShapeDtypeStruct((n, dim), jnp.float32),
    )(aggp1, degp, r1)

    h_pad = jnp.pad(h, ((0, n3 - n), (0, 0)))
    segsum = _make_segsum(n3, n_chunks, n, npad, with_deg=False,
                          stage_table=False)
    (aggp2,) = segsum(h_pad, ei_p, ones1)

    logp, out = pl.pallas_call(
        _layer2_body,
        out_shape=(jax.ShapeDtypeStruct((n, c_out), jnp.float32),
                   jax.ShapeDtypeStruct((n, c_out), jnp.float32)),
    )(aggp2, degp, h, w2l_t, w2r_t, b2)
    return (logp, out)


# speedup vs baseline: 1.0701x; 1.0701x over previous
import functools

import jax
import jax.numpy as jnp
from jax import lax
from jax.experimental import pallas as pl
from jax.experimental.pallas import tpu as pltpu
from jax.experimental.pallas import tpu_sc as plsc

NC = 2
NS = 16
NW = NC * NS
CH = 128
SEG = 20


def _proj1_body(x_ref, w_ref, b_ref, p_ref, r_ref):
    n = x_ref.shape[0]
    res = jnp.dot(x_ref[...], w_ref[...], preferred_element_type=jnp.float32)
    p_ref[:n, :] = res[:, :16]
    r_ref[:n, :] = res[:, 16:] + b_ref[...]


def _make_epilogue(n4_rows):
    rows_pw = n4_rows // NW
    mesh = plsc.VectorSubcoreMesh(core_axis_name="c", subcore_axis_name="s")
    scratch = [
        pltpu.VMEM((rows_pw, 16), jnp.float32),
        pltpu.VMEM((rows_pw, 16), jnp.float32),
        pltpu.VMEM((rows_pw, 16), jnp.float32),
        pltpu.VMEM((rows_pw, 16), jnp.float32),
        pltpu.VMEM((rows_pw,), jnp.float32),
        pltpu.VMEM((rows_pw,), jnp.float32),
    ]

    def body(aggp_h, degp_h, r1_h, h_out, a0v, a1v, r1v, hv, d0v, d1v):
        wid = lax.axis_index("s") * NC + lax.axis_index("c")
        w0 = wid * rows_pw
        pltpu.sync_copy(aggp_h.at[0, pl.ds(w0, rows_pw)], a0v)
        pltpu.sync_copy(aggp_h.at[1, pl.ds(w0, rows_pw)], a1v)
        pltpu.sync_copy(degp_h.at[0, pl.ds(w0, rows_pw)], d0v)
        pltpu.sync_copy(degp_h.at[1, pl.ds(w0, rows_pw)], d1v)
        pltpu.sync_copy(r1_h.at[pl.ds(w0, rows_pw)], r1v)

        def grp(i, c2):
            dv = d0v[pl.ds(i * 16, 16)] + d1v[pl.ds(i * 16, 16)]
            rec = 1.0 / jnp.maximum(dv, 1.0)
            for j in range(16):
                r = i * 16 + j
                av = a0v[r, :] + a1v[r, :]
                hv[r, :] = jnp.maximum(av * rec[j] + r1v[r, :], 0.0)
            return c2

        lax.fori_loop(0, rows_pw // 16, grp, 0)
        pltpu.sync_copy(hv, h_out.at[pl.ds(w0, rows_pw)])

    return pl.kernel(
        body, out_type=[jax.ShapeDtypeStruct((n4_rows, 16), jnp.float32)],
        mesh=mesh, scratch_types=scratch,
        compiler_params=pltpu.CompilerParams(use_tc_tiling_on_sc=False))


def _recip_cols(degp_ref, n, width):
    deg2 = degp_ref[0:1, :n] + degp_ref[1:2, :n]
    rec = 1.0 / jnp.maximum(deg2, 1.0)
    return jax.lax.dot_general(
        rec, jnp.ones((1, width), jnp.float32),
        (((0,), (0,)), ((), ())), preferred_element_type=jnp.float32)


def _layer1_epilogue_body(aggp_ref, degp_ref, r_ref, h_ref):
    n = r_ref.shape[0]
    agg = aggp_ref[0, :n, :] + aggp_ref[1, :n, :]
    mean = agg * _recip_cols(degp_ref, n, agg.shape[1])
    h_ref[...] = jnp.maximum(mean + r_ref[...], 0.0)


def _layer2_body(aggp_ref, degp_ref, h_ref, wl_ref, wr_ref, b_ref,
                 logp_ref, out_ref):
    n = h_ref.shape[0]
    agg = aggp_ref[0, :n, :] + aggp_ref[1, :n, :]
    mean = agg * _recip_cols(degp_ref, n, agg.shape[1])
    out = (jnp.dot(mean, wl_ref[...], preferred_element_type=jnp.float32)
           + jnp.dot(h_ref[...], wr_ref[...],
                     preferred_element_type=jnp.float32)
           + b_ref[...])
    out_ref[...] = out
    shifted = out - jnp.max(out, axis=1, keepdims=True)
    logp_ref[...] = shifted - jnp.log(
        jnp.sum(jnp.exp(shifted), axis=1, keepdims=True))


def _make_segsum(n3_rows, n_chunks, n_real, n_pad_edges, with_deg,
                 stage_table):
    rows_pt = n3_rows // NS

    mesh = plsc.VectorSubcoreMesh(core_axis_name="c", subcore_axis_name="s")
    out_type = [jax.ShapeDtypeStruct((NC, n3_rows, 16), jnp.float32)]
    if with_deg:
        out_type.append(jax.ShapeDtypeStruct((NC, n3_rows), jnp.float32))
    n_segs = n_chunks // SEG
    scratch = [
        pltpu.VMEM((n_chunks, CH), jnp.int32),
        pltpu.VMEM((n_chunks, CH), jnp.int32),
        [pltpu.VMEM((SEG * CH, 16), jnp.float32) for _ in range(2)],
        pltpu.VMEM((rows_pt, 16), jnp.float32),
        pltpu.VMEM_SHARED((n3_rows, 16), jnp.float32),
        [pltpu.SemaphoreType.DMA for _ in range(2)],
        [pltpu.SemaphoreType.DMA for _ in range(2)],
    ]
    if stage_table:
        scratch.append(pltpu.VMEM_SHARED((n3_rows, 16), jnp.float32))
    if with_deg:
        scratch += [
            pltpu.VMEM((CH,), jnp.float32),
            pltpu.VMEM((-(-rows_pt // 16) * 16,), jnp.float32),
            pltpu.VMEM_SHARED((n3_rows,), jnp.float32),
            [pltpu.SemaphoreType.DMA for _ in range(2)],
        ]

    def body(table_h, ei_h, ones1_h, *rest):
        it = iter(rest)
        agg_out = next(it)
        deg_out = next(it) if with_deg else None
        srcv, dstv, bufs, bounce, agg_s, gsem, ssem = (
            next(it), next(it), next(it), next(it), next(it), next(it),
            next(it))
        table_sc = next(it) if stage_table else None
        if with_deg:
            ones1v, degv, deg_s, dsem = next(it), next(it), next(it), next(it)
        cid = lax.axis_index("c")
        sid = lax.axis_index("s")
        wid = sid * NC + cid
        t0 = sid * rows_pt
        if stage_table:
            pltpu.sync_copy(table_h.at[pl.ds(t0, rows_pt)], bounce)
            pltpu.sync_copy(bounce, table_sc.at[pl.ds(t0, rows_pt)])
            table_s = table_sc
        else:
            table_s = table_h
        zv = jnp.zeros((16,), jnp.float32)

        def _zrow(r, c2):
            bounce[r, :] = zv
            return c2

        lax.fori_loop(0, rows_pt, _zrow, 0)
        pltpu.sync_copy(bounce, agg_s.at[pl.ds(t0, rows_pt)])
        if with_deg:
            def _zdeg(i, c2):
                degv[pl.ds(i * 16, 16)] = zv
                return c2

            lax.fori_loop(0, degv.shape[0] // 16, _zdeg, 0)
            pltpu.sync_copy(degv.at[pl.ds(0, rows_pt)],
                            deg_s.at[pl.ds(t0, rows_pt)])
            pltpu.sync_copy(ones1_h, ones1v)
        pltpu.sync_copy(ei_h.at[0, wid], srcv)
        pltpu.sync_copy(ei_h.at[1, wid], dstv)
        pad_rows = n_pad_edges // CH
        if pad_rows:
            iot = lax.iota(jnp.int32, 16)
            dpatt = n_real + iot

            @pl.when(wid == NW - 1)
            def _():
                for r in range(n_chunks - pad_rows, n_chunks):
                    for j in range(CH // 16):
                        dstv[r, pl.ds(j * 16, 16)] = dpatt
                        srcv[r, pl.ds(j * 16, 16)] = (
                            ((r * CH + j * 16) * 131 + iot * 7) % n_real)
        plsc.subcore_barrier()

        def fire_gathers(s, b):
            for k in range(SEG):
                pltpu.async_copy(table_s.at[srcv.at[s * SEG + k]],
                                 bufs[b].at[pl.ds(k * CH, CH)], gsem[b])

        def drain(sem, src, dst):
            for _ in range(SEG):
                pltpu.make_async_copy(src, dst, sem).wait()

        def fire_scatters(s, b):
            for k in range(SEG):
                c = s * SEG + k
                pltpu.async_copy(bufs[b].at[pl.ds(k * CH, CH)],
                                 agg_s.at[dstv.at[c]], ssem[b], add=True)
                if with_deg:
                    pltpu.async_copy(ones1v, deg_s.at[dstv.at[c]],
                                     dsem[b], add=True)

        def drain_scatters(b):
            drain(ssem[b], bufs[b].at[pl.ds(0, CH)], agg_s.at[dstv.at[0]])
            if with_deg:
                drain(dsem[b], ones1v, deg_s.at[dstv.at[0]])

        fire_gathers(0, 0)
        for s in range(n_segs):
            b = s % 2
            if s > 0:
                drain_scatters(1 - b)
            if s + 1 < n_segs:
                fire_gathers(s + 1, 1 - b)
            drain(gsem[b], table_s.at[srcv.at[0]], bufs[b].at[pl.ds(0, CH)])
            fire_scatters(s, b)
        drain_scatters((n_segs - 1) % 2)
        plsc.subcore_barrier()
        pltpu.sync_copy(agg_s.at[pl.ds(t0, rows_pt)], bounce)
        pltpu.sync_copy(bounce, agg_out.at[cid, pl.ds(t0, rows_pt)])
        if with_deg:
            pltpu.sync_copy(deg_s.at[pl.ds(t0, rows_pt)],
                            degv.at[pl.ds(0, rows_pt)])
            pltpu.sync_copy(degv.at[pl.ds(0, rows_pt)],
                            deg_out.at[cid, pl.ds(t0, rows_pt)])

    return pl.kernel(body, out_type=out_type, mesh=mesh,
                     scratch_types=scratch,
                     compiler_params=pltpu.CompilerParams(
                         use_tc_tiling_on_sc=False))


def kernel(x, edge_index, W1l, b1l, W1r, W2l, b2l, W2r):
    n, f_in = x.shape
    dim = W1l.shape[0]
    c_out = W2l.shape[0]
    e = edge_index.shape[1]
    assert dim == 16

    n3 = -(-(n + 16) // 256) * 256
    n_chunks = -(-e // (NW * CH))
    n_chunks = -(-n_chunks // SEG) * SEG
    npad = n_chunks * NW * CH - e

    ei_p = jnp.pad(edge_index, ((0, 0), (0, npad)),
                   constant_values=n).reshape(2, NW, n_chunks, CH)
    w1 = jnp.concatenate([W1l, W1r], axis=0).T
    b1 = b1l.reshape(1, dim)
    w2l_t = W2l.T
    w2r_t = W2r.T
    b2 = b2l.reshape(1, c_out)
    ones1 = jnp.ones((CH,), jnp.float32)

    p1_pad, r1_pad = pl.pallas_call(
        _proj1_body,
        out_shape=(jax.ShapeDtypeStruct((n3, dim), jnp.float32),
                   jax.ShapeDtypeStruct((n3, dim), jnp.float32)),
    )(x, w1, b1)

    segsum_deg = _make_segsum(n3, n_chunks, n, npad, with_deg=True,
                              stage_table=False)
    aggp1, degp = segsum_deg(p1_pad, ei_p, ones1)

    (h_pad,) = _make_epilogue(n3)(aggp1, degp, r1_pad)

    segsum = _make_segsum(n3, n_chunks, n, npad, with_deg=False,
                          stage_table=False)
    (aggp2,) = segsum(h_pad, ei_p, ones1)
    h = h_pad[:n]

    logp, out = pl.pallas_call(
        _layer2_body,
        out_shape=(jax.ShapeDtypeStruct((n, c_out), jnp.float32),
                   jax.ShapeDtypeStruct((n, c_out), jnp.float32)),
    )(aggp2, degp, h, w2l_t, w2r_t, b2)
    return (logp, out)
